# baseline (speedup 1.0000x reference)
import functools

import jax
import jax.numpy as jnp
from jax import lax
from jax.experimental import pallas as pl
from jax.experimental.pallas import tpu as pltpu
from jax.experimental.pallas import tpu_sc as plsc

_N = 10000
_D = 128
_L = 4
_OUT = 64
_CHUNK = 128
_NSUB = 16
_NACC = 10240
_ROWS_PER_TILE = _NACC // _NSUB


_KB = 16


def _spmm_kernel(ch_per_tile):
    mesh = plsc.VectorSubcoreMesh(core_axis_name="c", subcore_axis_name="s",
                                  num_cores=2, num_subcores=_NSUB)
    out_type = jax.ShapeDtypeStruct((2 * _NACC, _D), jnp.float32)
    scratch = (
        pltpu.VMEM((2, _CHUNK, _D), jnp.float32),
        pltpu.VMEM((2, _CHUNK), jnp.int32),
        pltpu.VMEM((2, _CHUNK), jnp.int32),
        pltpu.VMEM_SHARED((_NACC, _D), jnp.float32),
        pltpu.SemaphoreType.DMA,
        pltpu.SemaphoreType.DMA,
    )

    def body(x_hbm, g_hbm, s_hbm, zr_hbm, out_hbm,
             rows_v, gidx_v, sidx_v, acc, gsem, ssem):
        c = lax.axis_index("c")
        s = lax.axis_index("s")
        base = s * _ROWS_PER_TILE
        pltpu.sync_copy(zr_hbm, acc.at[pl.ds(base, _ROWS_PER_TILE)])
        plsc.subcore_barrier()
        ebase = c * (ch_per_tile * _NSUB * _CHUNK) + s * (ch_per_tile * _CHUNK)

        def pair(p, carry):
            off = ebase + 2 * p * _CHUNK
            pltpu.sync_copy(g_hbm.at[pl.ds(off, _CHUNK)], gidx_v.at[0])
            pltpu.sync_copy(g_hbm.at[pl.ds(off + _CHUNK, _CHUNK)], gidx_v.at[1])
            pltpu.sync_copy(s_hbm.at[pl.ds(off, _CHUNK)], sidx_v.at[0])
            pltpu.sync_copy(s_hbm.at[pl.ds(off + _CHUNK, _CHUNK)], sidx_v.at[1])
            pltpu.async_copy(x_hbm.at[gidx_v.at[0]], rows_v.at[0], gsem).wait()
            h0 = pltpu.async_copy(rows_v.at[0], acc.at[sidx_v.at[0]], ssem,
                                  add=True)
            pltpu.async_copy(x_hbm.at[gidx_v.at[1]], rows_v.at[1], gsem).wait()
            h0.wait()
            pltpu.async_copy(rows_v.at[1], acc.at[sidx_v.at[1]], ssem,
                             add=True).wait()
            return carry

        lax.fori_loop(0, ch_per_tile // 2, pair, 0)
        plsc.subcore_barrier()
        obase = c * _NACC + base
        pltpu.sync_copy(acc.at[pl.ds(base, _ROWS_PER_TILE)],
                        out_hbm.at[pl.ds(obase, _ROWS_PER_TILE)])

    return pl.kernel(body, out_type=out_type, mesh=mesh,
                     scratch_types=scratch)


def _tc_body(*refs, first, last):
    k = 0
    x = refs[k][...]; k += 1
    sin = refs[k][...]; k += 1
    sout = refs[k][...]; k += 1
    din = refs[k][...]; k += 1
    dout = refs[k][...]; k += 1
    if not first:
        fused_in = refs[k][...]; k += 1
    wselfT = refs[k][...]; k += 1
    bself = refs[k][...]; k += 1
    w1T = refs[k][...]; k += 1
    wdT = refs[k][...]; k += 1
    bctx = refs[k][...]; k += 1
    a1T = refs[k][...]; k += 1
    b1 = refs[k][...]; k += 1
    a2 = refs[k][...]; k += 1
    lng = refs[k][...]; k += 1
    lnb = refs[k][...]; k += 1
    fwl = refs[k][...]; k += 1
    if last:
        wclsT = refs[k][...]; k += 1
        bcls = refs[k][...]; k += 1

    f32 = jnp.float32
    dinc = din[:, 0:1]
    doutc = dout[:, 0:1]
    p = jnp.dot(x, w1T, preferred_element_type=f32)
    sv = jnp.dot(x, wselfT, preferred_element_type=f32) + bself
    outg = dinc * p + jnp.dot(sin, wdT, preferred_element_type=f32) + bctx
    inc = doutc * p + jnp.dot(sout, wdT, preferred_element_type=f32) + bctx

    def score(v):
        t = jnp.tanh(jnp.dot(v, a1T, preferred_element_type=f32) + b1)
        return jnp.sum(t * a2, axis=-1, keepdims=True)

    s0, s1, s2 = score(sv), score(outg), score(inc)
    m = jnp.maximum(jnp.maximum(s0, s1), s2)
    e0 = jnp.exp(s0 - m)
    e1 = jnp.exp(s1 - m)
    e2 = jnp.exp(s2 - m)
    h = (e0 * sv + e1 * outg + e2 * inc) / (e0 + e1 + e2)
    mu = jnp.mean(h, axis=-1, keepdims=True)
    var = jnp.mean((h - mu) ** 2, axis=-1, keepdims=True)
    hn = (h - mu) * lax.rsqrt(var + 1e-5) * lng + lnb
    hr = jnp.maximum(hn, 0.0)
    cur = hr if first else hr + x
    f = fwl * cur if first else fused_in + fwl * cur
    if last:
        refs[-1][...] = jnp.dot(f, wclsT, preferred_element_type=f32) + bcls
    else:
        refs[-2][...] = cur
        refs[-1][...] = f


def _tc_layer(x, sin, sout, din, dout, fused, w, first, last):
    bn = min(400, _N)
    grid = (_N // bn,)

    def rowspec(a):
        return pl.BlockSpec((bn, a.shape[1]), lambda i: (i, 0))

    def fullspec(a):
        return pl.BlockSpec(a.shape, lambda i: (0,) * a.ndim)

    operands = [x, sin, sout, din, dout]
    specs = [rowspec(a) for a in operands]
    if not first:
        operands.append(fused)
        specs.append(rowspec(fused))
    operands += w
    specs += [fullspec(a) for a in w]
    if last:
        out_shape = jax.ShapeDtypeStruct((_N, _OUT), jnp.float32)
        out_specs = pl.BlockSpec((bn, _OUT), lambda i: (i, 0))
    else:
        out_shape = (jax.ShapeDtypeStruct((_N, _D), jnp.float32),
                     jax.ShapeDtypeStruct((_N, _D), jnp.float32))
        out_specs = (pl.BlockSpec((bn, _D), lambda i: (i, 0)),
                     pl.BlockSpec((bn, _D), lambda i: (i, 0)))
    return pl.pallas_call(
        functools.partial(_tc_body, first=first, last=last),
        grid=grid, in_specs=specs, out_specs=out_specs,
        out_shape=out_shape)(*operands)


def kernel(node_features, edge_index, Wself, bself, Wctx, bctx, A1, b1, A2,
           ln_g, ln_b, fusion_w, Wcls, bcls):
    e = edge_index.shape[1]
    ch_per_tile = -(-e // (_NSUB * _CHUNK))
    ch_per_tile += ch_per_tile % 2
    e_pad = ch_per_tile * _NSUB * _CHUNK
    pad = e_pad - e
    src = edge_index[0]
    dst = edge_index[1]
    zi = jnp.zeros((pad,), jnp.int32)
    di = jnp.full((pad,), _N, jnp.int32)
    gmat = jnp.concatenate([src, zi, dst, zi])
    smat = jnp.concatenate([dst, di, src, di])
    zr = jnp.zeros((_ROWS_PER_TILE, _D), jnp.float32)
    ones_rows = jnp.ones((_N, _D), jnp.float32)
    fw = jax.nn.softmax(fusion_w)

    spmm = _spmm_kernel(ch_per_tile)

    deg = spmm(ones_rows, gmat, smat, zr)
    din = deg[:_N]
    dout = deg[_NACC:_NACC + _N]

    cur = node_features
    fused = None
    out = None
    for l in range(_L):
        s_acc = spmm(cur, gmat, smat, zr)
        sin = s_acc[:_N]
        sout = s_acc[_NACC:_NACC + _N]
        w = [
            Wself[l].T, bself[l][None, :],
            Wctx[l, :, :_D].T, (Wctx[l, :, _D:] - Wctx[l, :, :_D]).T,
            bctx[l][None, :],
            A1[l].T, b1[l][None, :], A2[l],
            ln_g[l][None, :], ln_b[l][None, :],
            jnp.broadcast_to(fw[l], (1, _D)),
        ]
        first = l == 0
        last = l == _L - 1
        if last:
            w += [Wcls.T, bcls[None, :]]
        res = _tc_layer(cur, sin, sout, din, dout, fused, w, first, last)
        if last:
            out = res
        else:
            cur, fused = res
    return out


# speedup vs baseline: 1.0121x; 1.0121x over previous
import functools

import jax
import jax.numpy as jnp
from jax import lax
from jax.experimental import pallas as pl
from jax.experimental.pallas import tpu as pltpu
from jax.experimental.pallas import tpu_sc as plsc

_N = 10000
_D = 128
_L = 4
_OUT = 64
_CHUNK = 128
_NSUB = 16
_NACC = 10240
_ROWS_PER_TILE = _NACC // _NSUB


_KB = 16


def _spmm_kernel(ch_per_tile):
    mesh = plsc.VectorSubcoreMesh(core_axis_name="c", subcore_axis_name="s",
                                  num_cores=2, num_subcores=_NSUB)
    out_type = jax.ShapeDtypeStruct((2 * _NACC, _D), jnp.float32)
    scratch = (
        pltpu.VMEM((_CHUNK, _D), jnp.float32),
        pltpu.VMEM((_CHUNK,), jnp.int32),
        pltpu.VMEM((_CHUNK,), jnp.int32),
        pltpu.VMEM_SHARED((_NACC, _D), jnp.float32),
        pltpu.SemaphoreType.DMA,
    )

    def body(x_hbm, g_hbm, s_hbm, zr_hbm, out_hbm,
             rows_v, gidx_v, sidx_v, acc, sem):
        c = lax.axis_index("c")
        s = lax.axis_index("s")
        base = s * _ROWS_PER_TILE
        pltpu.sync_copy(zr_hbm, acc.at[pl.ds(base, _ROWS_PER_TILE)])
        plsc.subcore_barrier()
        ebase = c * (ch_per_tile * _NSUB * _CHUNK) + s * (ch_per_tile * _CHUNK)

        def step(i, carry):
            off = ebase + i * _CHUNK
            pltpu.sync_copy(g_hbm.at[pl.ds(off, _CHUNK)], gidx_v)
            pltpu.sync_copy(s_hbm.at[pl.ds(off, _CHUNK)], sidx_v)
            pltpu.async_copy(x_hbm.at[gidx_v], rows_v, sem).wait()
            pltpu.sync_copy(rows_v, acc.at[sidx_v], add=True)
            return carry

        lax.fori_loop(0, ch_per_tile, step, 0)
        plsc.subcore_barrier()
        obase = c * _NACC + base
        pltpu.sync_copy(acc.at[pl.ds(base, _ROWS_PER_TILE)],
                        out_hbm.at[pl.ds(obase, _ROWS_PER_TILE)])

    return pl.kernel(body, out_type=out_type, mesh=mesh,
                     scratch_types=scratch)


def _tc_body(*refs, first, last):
    k = 0
    x = refs[k][...]; k += 1
    sin = refs[k][...]; k += 1
    sout = refs[k][...]; k += 1
    din = refs[k][...]; k += 1
    dout = refs[k][...]; k += 1
    if not first:
        fused_in = refs[k][...]; k += 1
    wselfT = refs[k][...]; k += 1
    bself = refs[k][...]; k += 1
    w1T = refs[k][...]; k += 1
    wdT = refs[k][...]; k += 1
    bctx = refs[k][...]; k += 1
    a1T = refs[k][...]; k += 1
    b1 = refs[k][...]; k += 1
    a2 = refs[k][...]; k += 1
    lng = refs[k][...]; k += 1
    lnb = refs[k][...]; k += 1
    fwl = refs[k][...]; k += 1
    if last:
        wclsT = refs[k][...]; k += 1
        bcls = refs[k][...]; k += 1

    f32 = jnp.float32
    dinc = din[:, 0:1]
    doutc = dout[:, 0:1]
    p = jnp.dot(x, w1T, preferred_element_type=f32)
    sv = jnp.dot(x, wselfT, preferred_element_type=f32) + bself
    outg = dinc * p + jnp.dot(sin, wdT, preferred_element_type=f32) + bctx
    inc = doutc * p + jnp.dot(sout, wdT, preferred_element_type=f32) + bctx

    def score(v):
        t = jnp.tanh(jnp.dot(v, a1T, preferred_element_type=f32) + b1)
        return jnp.sum(t * a2, axis=-1, keepdims=True)

    s0, s1, s2 = score(sv), score(outg), score(inc)
    m = jnp.maximum(jnp.maximum(s0, s1), s2)
    e0 = jnp.exp(s0 - m)
    e1 = jnp.exp(s1 - m)
    e2 = jnp.exp(s2 - m)
    h = (e0 * sv + e1 * outg + e2 * inc) / (e0 + e1 + e2)
    mu = jnp.mean(h, axis=-1, keepdims=True)
    var = jnp.mean((h - mu) ** 2, axis=-1, keepdims=True)
    hn = (h - mu) * lax.rsqrt(var + 1e-5) * lng + lnb
    hr = jnp.maximum(hn, 0.0)
    cur = hr if first else hr + x
    f = fwl * cur if first else fused_in + fwl * cur
    if last:
        refs[-1][...] = jnp.dot(f, wclsT, preferred_element_type=f32) + bcls
    else:
        refs[-2][...] = cur
        refs[-1][...] = f


def _tc_layer(x, sin, sout, din, dout, fused, w, first, last):
    bn = min(400, _N)
    grid = (_N // bn,)

    def rowspec(a):
        return pl.BlockSpec((bn, a.shape[1]), lambda i: (i, 0))

    def fullspec(a):
        return pl.BlockSpec(a.shape, lambda i: (0,) * a.ndim)

    operands = [x, sin, sout, din, dout]
    specs = [rowspec(a) for a in operands]
    if not first:
        operands.append(fused)
        specs.append(rowspec(fused))
    operands += w
    specs += [fullspec(a) for a in w]
    if last:
        out_shape = jax.ShapeDtypeStruct((_N, _OUT), jnp.float32)
        out_specs = pl.BlockSpec((bn, _OUT), lambda i: (i, 0))
    else:
        out_shape = (jax.ShapeDtypeStruct((_N, _D), jnp.float32),
                     jax.ShapeDtypeStruct((_N, _D), jnp.float32))
        out_specs = (pl.BlockSpec((bn, _D), lambda i: (i, 0)),
                     pl.BlockSpec((bn, _D), lambda i: (i, 0)))
    return pl.pallas_call(
        functools.partial(_tc_body, first=first, last=last),
        grid=grid, in_specs=specs, out_specs=out_specs,
        out_shape=out_shape)(*operands)


def kernel(node_features, edge_index, Wself, bself, Wctx, bctx, A1, b1, A2,
           ln_g, ln_b, fusion_w, Wcls, bcls):
    e = edge_index.shape[1]
    ch_per_tile = -(-e // (_NSUB * _CHUNK))
    ch_per_tile += ch_per_tile % 2
    e_pad = ch_per_tile * _NSUB * _CHUNK
    pad = e_pad - e
    src = edge_index[0]
    dst = edge_index[1]
    zi = jnp.zeros((pad,), jnp.int32)
    di = jnp.full((pad,), _N, jnp.int32)
    gmat = jnp.concatenate([src, zi, dst, zi])
    smat = jnp.concatenate([dst, di, src, di])
    zr = jnp.zeros((_ROWS_PER_TILE, _D), jnp.float32)
    ones_rows = jnp.ones((_N, _D), jnp.float32)
    fw = jax.nn.softmax(fusion_w)

    spmm = _spmm_kernel(ch_per_tile)

    lin = jnp.arange(e_pad, dtype=jnp.int32) % _N
    deg = spmm(ones_rows, jnp.concatenate([lin, lin]), smat, zr)
    din = deg[:_N]
    dout = deg[_NACC:_NACC + _N]

    cur = node_features
    fused = None
    out = None
    for l in range(_L):
        s_acc = spmm(cur, gmat, smat, zr)
        sin = s_acc[:_N]
        sout = s_acc[_NACC:_NACC + _N]
        w = [
            Wself[l].T, bself[l][None, :],
            Wctx[l, :, :_D].T, (Wctx[l, :, _D:] - Wctx[l, :, :_D]).T,
            bctx[l][None, :],
            A1[l].T, b1[l][None, :], A2[l],
            ln_g[l][None, :], ln_b[l][None, :],
            jnp.broadcast_to(fw[l], (1, _D)),
        ]
        first = l == 0
        last = l == _L - 1
        if last:
            w += [Wcls.T, bcls[None, :]]
        res = _tc_layer(cur, sin, sout, din, dout, fused, w, first, last)
        if last:
            out = res
        else:
            cur, fused = res
    return out


# speedup vs baseline: 1.1656x; 1.1517x over previous
import functools

import jax
import jax.numpy as jnp
from jax import lax
from jax.experimental import pallas as pl
from jax.experimental.pallas import tpu as pltpu
from jax.experimental.pallas import tpu_sc as plsc

_N = 10000
_D = 128
_L = 4
_OUT = 64
_CHUNK = 128
_NSUB = 16
_NACC = 10240
_ROWS_PER_TILE = _NACC // _NSUB


_KB = 16


def _spmm_kernel(ch_per_tile):
    mesh = plsc.VectorSubcoreMesh(core_axis_name="c", subcore_axis_name="s",
                                  num_cores=2, num_subcores=_NSUB)
    out_type = jax.ShapeDtypeStruct((2 * _NACC, _D), jnp.float32)
    scratch = (
        pltpu.VMEM((_CHUNK, _D), jnp.float32),
        pltpu.VMEM((_CHUNK,), jnp.int32),
        pltpu.VMEM((_CHUNK,), jnp.int32),
        pltpu.VMEM_SHARED((_NACC, _D), jnp.float32),
        pltpu.SemaphoreType.DMA,
    )

    def body(x_hbm, g_hbm, s_hbm, zr_hbm, out_hbm,
             rows_v, gidx_v, sidx_v, acc, sem):
        c = lax.axis_index("c")
        s = lax.axis_index("s")
        base = s * _ROWS_PER_TILE
        pltpu.sync_copy(zr_hbm, acc.at[pl.ds(base, _ROWS_PER_TILE)])
        plsc.subcore_barrier()
        ebase = c * (ch_per_tile * _NSUB * _CHUNK) + s * (ch_per_tile * _CHUNK)

        def step(i, carry):
            off = ebase + i * _CHUNK
            pltpu.sync_copy(g_hbm.at[pl.ds(off, _CHUNK)], gidx_v)
            pltpu.sync_copy(s_hbm.at[pl.ds(off, _CHUNK)], sidx_v)
            pltpu.async_copy(x_hbm.at[gidx_v], rows_v, sem).wait()
            pltpu.sync_copy(rows_v, acc.at[sidx_v], add=True)
            return carry

        lax.fori_loop(0, ch_per_tile, step, 0)
        plsc.subcore_barrier()
        obase = c * _NACC + base
        pltpu.sync_copy(acc.at[pl.ds(base, _ROWS_PER_TILE)],
                        out_hbm.at[pl.ds(obase, _ROWS_PER_TILE)])

    return pl.kernel(body, out_type=out_type, mesh=mesh,
                     scratch_types=scratch)


def _tc_body(*refs, first, last):
    k = 0
    x = refs[k][...]; k += 1
    sin = refs[k][...]; k += 1
    sout = refs[k][...]; k += 1
    din = refs[k][...]; k += 1
    dout = refs[k][...]; k += 1
    if not first:
        fused_in = refs[k][...]; k += 1
    wselfT = refs[k][...]; k += 1
    bself = refs[k][...]; k += 1
    w1T = refs[k][...]; k += 1
    wdT = refs[k][...]; k += 1
    bctx = refs[k][...]; k += 1
    a1T = refs[k][...]; k += 1
    b1 = refs[k][...]; k += 1
    a2 = refs[k][...]; k += 1
    lng = refs[k][...]; k += 1
    lnb = refs[k][...]; k += 1
    fwl = refs[k][...]; k += 1
    if last:
        wclsT = refs[k][...]; k += 1
        bcls = refs[k][...]; k += 1

    f32 = jnp.float32
    dinc = din[:, 0:1]
    doutc = dout[:, 0:1]
    p = jnp.dot(x, w1T, preferred_element_type=f32)
    sv = jnp.dot(x, wselfT, preferred_element_type=f32) + bself
    outg = dinc * p + jnp.dot(sin, wdT, preferred_element_type=f32) + bctx
    inc = doutc * p + jnp.dot(sout, wdT, preferred_element_type=f32) + bctx

    def score(v):
        t = jnp.tanh(jnp.dot(v, a1T, preferred_element_type=f32) + b1)
        return jnp.sum(t * a2, axis=-1, keepdims=True)

    s0, s1, s2 = score(sv), score(outg), score(inc)
    m = jnp.maximum(jnp.maximum(s0, s1), s2)
    e0 = jnp.exp(s0 - m)
    e1 = jnp.exp(s1 - m)
    e2 = jnp.exp(s2 - m)
    h = (e0 * sv + e1 * outg + e2 * inc) / (e0 + e1 + e2)
    mu = jnp.mean(h, axis=-1, keepdims=True)
    var = jnp.mean((h - mu) ** 2, axis=-1, keepdims=True)
    hn = (h - mu) * lax.rsqrt(var + 1e-5) * lng + lnb
    hr = jnp.maximum(hn, 0.0)
    cur = hr if first else hr + x
    f = fwl * cur if first else fused_in + fwl * cur
    if last:
        refs[-1][...] = jnp.dot(f, wclsT, preferred_element_type=f32) + bcls
    else:
        refs[-2][...] = cur
        refs[-1][...] = f


def _tc_layer(x, sin, sout, din, dout, fused, w, first, last):
    bn = min(400, _N)
    grid = (_N // bn,)

    def rowspec(a):
        return pl.BlockSpec((bn, a.shape[1]), lambda i: (i, 0))

    def fullspec(a):
        return pl.BlockSpec(a.shape, lambda i: (0,) * a.ndim)

    operands = [x, sin, sout, din, dout]
    specs = [rowspec(a) for a in operands]
    if not first:
        operands.append(fused)
        specs.append(rowspec(fused))
    operands += w
    specs += [fullspec(a) for a in w]
    if last:
        out_shape = jax.ShapeDtypeStruct((_N, _OUT), jnp.float32)
        out_specs = pl.BlockSpec((bn, _OUT), lambda i: (i, 0))
    else:
        out_shape = (jax.ShapeDtypeStruct((_N, _D), jnp.float32),
                     jax.ShapeDtypeStruct((_N, _D), jnp.float32))
        out_specs = (pl.BlockSpec((bn, _D), lambda i: (i, 0)),
                     pl.BlockSpec((bn, _D), lambda i: (i, 0)))
    return pl.pallas_call(
        functools.partial(_tc_body, first=first, last=last),
        grid=grid, in_specs=specs, out_specs=out_specs,
        out_shape=out_shape)(*operands)


def kernel(node_features, edge_index, Wself, bself, Wctx, bctx, A1, b1, A2,
           ln_g, ln_b, fusion_w, Wcls, bcls):
    e = edge_index.shape[1]
    ch_per_tile = -(-e // (_NSUB * _CHUNK))
    e_pad = ch_per_tile * _NSUB * _CHUNK
    pad = e_pad - e
    src = edge_index[0]
    dst = edge_index[1]
    zi = jnp.zeros((pad,), jnp.int32)
    di = jnp.full((pad,), _N, jnp.int32)
    gmat = jnp.concatenate([src, zi, dst, zi])
    smat = jnp.concatenate([dst, di, src, di])
    zr = jnp.zeros((_ROWS_PER_TILE, _D), jnp.float32)
    ones_rows = jnp.ones((_N, _D), jnp.float32)
    fw = jax.nn.softmax(fusion_w)

    spmm = _spmm_kernel(ch_per_tile)

    deg = spmm(ones_rows, gmat, smat, zr)
    din = deg[:_N]
    dout = deg[_NACC:_NACC + _N]

    cur = node_features
    fused = None
    out = None
    for l in range(_L):
        s_acc = spmm(cur, gmat, smat, zr)
        sin = s_acc[:_N]
        sout = s_acc[_NACC:_NACC + _N]
        w = [
            Wself[l].T, bself[l][None, :],
            Wctx[l, :, :_D].T, (Wctx[l, :, _D:] - Wctx[l, :, :_D]).T,
            bctx[l][None, :],
            A1[l].T, b1[l][None, :], A2[l],
            ln_g[l][None, :], ln_b[l][None, :],
            jnp.broadcast_to(fw[l], (1, _D)),
        ]
        first = l == 0
        last = l == _L - 1
        if last:
            w += [Wcls.T, bcls[None, :]]
        res = _tc_layer(cur, sin, sout, din, dout, fused, w, first, last)
        if last:
            out = res
        else:
            cur, fused = res
    return out


# speedup vs baseline: 1.1658x; 1.0002x over previous
import functools

import jax
import jax.numpy as jnp
from jax import lax
from jax.experimental import pallas as pl
from jax.experimental.pallas import tpu as pltpu
from jax.experimental.pallas import tpu_sc as plsc

_N = 10000
_D = 128
_L = 4
_OUT = 64
_CHUNK = 128
_NSUB = 16
_NACC = 10240
_ROWS_PER_TILE = _NACC // _NSUB


def _spmm_kernel(ch_per_tile):
    mesh = plsc.VectorSubcoreMesh(core_axis_name="c", subcore_axis_name="s",
                                  num_cores=2, num_subcores=_NSUB)
    out_type = jax.ShapeDtypeStruct((2 * _NACC, _D), jnp.float32)
    scratch = (
        pltpu.VMEM((_CHUNK, _D), jnp.float32),
        pltpu.VMEM((_CHUNK,), jnp.int32),
        pltpu.VMEM((_CHUNK,), jnp.int32),
        pltpu.VMEM_SHARED((_NACC, _D), jnp.float32),
        pltpu.SemaphoreType.DMA,
    )

    def body(x_hbm, g_hbm, s_hbm, zr_hbm, out_hbm,
             rows_v, gidx_v, sidx_v, acc, sem):
        c = lax.axis_index("c")
        s = lax.axis_index("s")
        base = s * _ROWS_PER_TILE
        pltpu.sync_copy(zr_hbm, acc.at[pl.ds(base, _ROWS_PER_TILE)])
        plsc.subcore_barrier()
        ebase = c * (ch_per_tile * _NSUB * _CHUNK) + s * (ch_per_tile * _CHUNK)

        def step(i, carry):
            off = ebase + i * _CHUNK
            pltpu.sync_copy(g_hbm.at[pl.ds(off, _CHUNK)], gidx_v)
            pltpu.sync_copy(s_hbm.at[pl.ds(off, _CHUNK)], sidx_v)
            pltpu.async_copy(x_hbm.at[gidx_v], rows_v, sem).wait()
            pltpu.sync_copy(rows_v, acc.at[sidx_v], add=True)
            return carry

        lax.fori_loop(0, ch_per_tile, step, 0)
        plsc.subcore_barrier()
        obase = c * _NACC + base
        pltpu.sync_copy(acc.at[pl.ds(base, _ROWS_PER_TILE)],
                        out_hbm.at[pl.ds(obase, _ROWS_PER_TILE)])

    return pl.kernel(body, out_type=out_type, mesh=mesh,
                     scratch_types=scratch)


def _tc_body(*refs, first, last):
    k = 0
    x = refs[k][...]; k += 1
    sin = refs[k][...]; k += 1
    sout = refs[k][...]; k += 1
    din = refs[k][...]; k += 1
    dout = refs[k][...]; k += 1
    if not first:
        fused_in = refs[k][...]; k += 1
    wselfT = refs[k][...]; k += 1
    bself = refs[k][...]; k += 1
    w1T = refs[k][...]; k += 1
    wdT = refs[k][...]; k += 1
    bctx = refs[k][...]; k += 1
    a1T = refs[k][...]; k += 1
    b1 = refs[k][...]; k += 1
    a2 = refs[k][...]; k += 1
    lng = refs[k][...]; k += 1
    lnb = refs[k][...]; k += 1
    fwl = refs[k][...]; k += 1
    if last:
        wclsT = refs[k][...]; k += 1
        bcls = refs[k][...]; k += 1

    f32 = jnp.float32
    dinc = din[:, 0:1]
    doutc = dout[:, 0:1]
    p = jnp.dot(x, w1T, preferred_element_type=f32)
    sv = jnp.dot(x, wselfT, preferred_element_type=f32) + bself
    outg = dinc * p + jnp.dot(sin, wdT, preferred_element_type=f32) + bctx
    inc = doutc * p + jnp.dot(sout, wdT, preferred_element_type=f32) + bctx

    def score(v):
        t = jnp.tanh(jnp.dot(v, a1T, preferred_element_type=f32) + b1)
        return jnp.sum(t * a2, axis=-1, keepdims=True)

    s0, s1, s2 = score(sv), score(outg), score(inc)
    m = jnp.maximum(jnp.maximum(s0, s1), s2)
    e0 = jnp.exp(s0 - m)
    e1 = jnp.exp(s1 - m)
    e2 = jnp.exp(s2 - m)
    h = (e0 * sv + e1 * outg + e2 * inc) / (e0 + e1 + e2)
    mu = jnp.mean(h, axis=-1, keepdims=True)
    var = jnp.mean((h - mu) ** 2, axis=-1, keepdims=True)
    hn = (h - mu) * lax.rsqrt(var + 1e-5) * lng + lnb
    hr = jnp.maximum(hn, 0.0)
    cur = hr if first else hr + x
    f = fwl * cur if first else fused_in + fwl * cur
    if last:
        refs[-1][...] = jnp.dot(f, wclsT, preferred_element_type=f32) + bcls
    else:
        refs[-2][...] = cur
        refs[-1][...] = f


def _tc_layer(x, sin, sout, din, dout, fused, w, first, last):
    bn = min(400, _N)
    grid = (_N // bn,)

    def rowspec(a):
        return pl.BlockSpec((bn, a.shape[1]), lambda i: (i, 0))

    def fullspec(a):
        return pl.BlockSpec(a.shape, lambda i: (0,) * a.ndim)

    operands = [x, sin, sout, din, dout]
    specs = [rowspec(a) for a in operands]
    if not first:
        operands.append(fused)
        specs.append(rowspec(fused))
    operands += w
    specs += [fullspec(a) for a in w]
    if last:
        out_shape = jax.ShapeDtypeStruct((_N, _OUT), jnp.float32)
        out_specs = pl.BlockSpec((bn, _OUT), lambda i: (i, 0))
    else:
        out_shape = (jax.ShapeDtypeStruct((_N, _D), jnp.float32),
                     jax.ShapeDtypeStruct((_N, _D), jnp.float32))
        out_specs = (pl.BlockSpec((bn, _D), lambda i: (i, 0)),
                     pl.BlockSpec((bn, _D), lambda i: (i, 0)))
    return pl.pallas_call(
        functools.partial(_tc_body, first=first, last=last),
        grid=grid, in_specs=specs, out_specs=out_specs,
        out_shape=out_shape)(*operands)


def kernel(node_features, edge_index, Wself, bself, Wctx, bctx, A1, b1, A2,
           ln_g, ln_b, fusion_w, Wcls, bcls):
    e = edge_index.shape[1]
    ch_per_tile = -(-e // (_NSUB * _CHUNK))
    e_pad = ch_per_tile * _NSUB * _CHUNK
    pad = e_pad - e
    src = edge_index[0]
    dst = edge_index[1]
    zi = jnp.zeros((pad,), jnp.int32)
    di = jnp.full((pad,), _N, jnp.int32)
    gmat = jnp.concatenate([src, zi, dst, zi])
    smat = jnp.concatenate([dst, di, src, di])
    zr = jnp.zeros((_ROWS_PER_TILE, _D), jnp.float32)
    ones_rows = jnp.ones((_N, _D), jnp.float32)
    fw = jax.nn.softmax(fusion_w)

    spmm = _spmm_kernel(ch_per_tile)

    deg = spmm(ones_rows, gmat, smat, zr)
    din = deg[:_N]
    dout = deg[_NACC:_NACC + _N]

    cur = node_features
    fused = None
    out = None
    for l in range(_L):
        s_acc = spmm(cur, gmat, smat, zr)
        sin = s_acc[:_N]
        sout = s_acc[_NACC:_NACC + _N]
        w = [
            Wself[l].T, bself[l][None, :],
            Wctx[l, :, :_D].T, (Wctx[l, :, _D:] - Wctx[l, :, :_D]).T,
            bctx[l][None, :],
            A1[l].T, b1[l][None, :], A2[l],
            ln_g[l][None, :], ln_b[l][None, :],
            jnp.broadcast_to(fw[l], (1, _D)),
        ]
        first = l == 0
        last = l == _L - 1
        if last:
            w += [Wcls.T, bcls[None, :]]
        res = _tc_layer(cur, sin, sout, din, dout, fused, w, first, last)
        if last:
            out = res
        else:
            cur, fused = res
    return out
